# baseline (speedup 1.0000x reference)
import functools

import jax
import jax.numpy as jnp
from jax import lax
from jax.experimental import pallas as pl
from jax.experimental.pallas import tpu as pltpu
from jax.experimental.pallas import tpu_sc as plsc

_NC = 2
_NS = 16
_L = 16
_F = 66
_D = 4
_RW = 8
_H1 = 16
_H2 = 8
_SPB = 64
_G = _SPB // _L

_OW1 = 0
_OB1 = _OW1 + _F * _D * _H1
_OW2 = _OB1 + _H1
_OB2 = _OW2 + _H2 * _H1
_OW3 = _OB2 + _H2
_OB3 = _OW3 + _H2
_WLEN = 4392


@functools.lru_cache(maxsize=None)
def _make_fused(B, n_idx):
    nw = _NC * _NS
    spw = B // nw
    n_chunks = spw // _SPB
    ch = _SPB * _F
    assert spw % _SPB == 0 and ch % 8 == 0 and n_chunks % 2 == 0

    mesh = plsc.VectorSubcoreMesh(
        core_axis_name="c", subcore_axis_name="s",
        num_cores=_NC, num_subcores=_NS)

    @functools.partial(
        pl.kernel,
        out_type=jax.ShapeDtypeStruct((B,), jnp.float32),
        mesh=mesh,
        scratch_types=[
            pltpu.VMEM((ch,), jnp.int32),
            pltpu.VMEM((ch,), jnp.int32),
            pltpu.VMEM((ch, _RW), jnp.float32),
            pltpu.VMEM((ch, _RW), jnp.float32),
            pltpu.VMEM((_WLEN,), jnp.float32),
            pltpu.VMEM(((_G * _H1 + _G) * _L,), jnp.float32),
            pltpu.VMEM((spw,), jnp.float32),
            pltpu.SemaphoreType.DMA((2,)),
        ],
        compiler_params=pltpu.CompilerParams(
            use_tc_tiling_on_sc=False, needs_layout_passes=False),
    )
    def fused_k(tab_hbm, idx_hbm, wpack_hbm, out_hbm,
                i0_v, i1_v, r0_v, r1_v, w_v, h1_v, out_v, gsem):
        wid = lax.axis_index("s") * _NC + lax.axis_index("c")
        sbase = wid * spw
        ibase = wid * spw * _F
        idx_bufs = (i0_v, i1_v)
        row_bufs = (r0_v, r1_v)

        pltpu.sync_copy(wpack_hbm, w_v)

        iota = lax.iota(jnp.int32, _L)
        rowbase = [(iota + g * _L) * _F for g in range(_G)]
        dcol = [jnp.full((_L,), d, jnp.int32) for d in range(_D + 1)]
        zero = jnp.zeros((_L,), jnp.float32)

        def start_gather_dyn(c_off, parity):
            pltpu.sync_copy(
                idx_hbm.at[pl.ds(ibase + c_off * ch, ch)], idx_bufs[parity])
            pltpu.async_copy(
                tab_hbm.at[idx_bufs[parity]], row_bufs[parity],
                gsem.at[parity])

        def wait_gather(parity):
            pltpu.make_async_copy(
                tab_hbm.at[pl.ds(0, ch)], row_bufs[parity],
                gsem.at[parity]).wait()

        def layer1(rv):
            for half in range(2):
                def f_body(f, carry, rv=rv, half=half):
                    accs = list(carry[:_G * 8])
                    baccs = list(carry[_G * 8:])
                    ws = []
                    for d in range(_D):
                        wv = w_v[pl.ds((f * _D + d) * _H1 + half * 8, _L)]
                        ws.append([wv[j8] for j8 in range(8)])
                    for g in range(_G):
                        idx0 = rowbase[g] + f
                        for d in range(_D):
                            xv = plsc.load_gather(rv, [idx0, dcol[d]])
                            for j8 in range(8):
                                accs[g * 8 + j8] = (
                                    accs[g * 8 + j8] + xv * ws[d][j8])
                        if half == 0:
                            baccs[g] = baccs[g] + plsc.load_gather(
                                rv, [idx0, dcol[_D]])
                    return tuple(accs) + tuple(baccs)

                n_b = _G if half == 0 else 0
                init = (zero,) * (_G * 8) + (zero,) * n_b
                out = lax.fori_loop(0, _F, f_body, init, unroll=1)
                for g in range(_G):
                    for j8 in range(8):
                        h1_v[pl.ds((g * _H1 + half * 8 + j8) * _L, _L)] = (
                            out[g * 8 + j8])
                if half == 0:
                    for g in range(_G):
                        h1_v[pl.ds((_G * _H1 + g) * _L, _L)] = out[_G * 8 + g]

        def tail(out_off):
            b1v = w_v[pl.ds(_OB1, _L)]
            b2v = w_v[pl.ds(_OB2, _L)]
            w3v = w_v[pl.ds(_OW3, _L)]

            def g_body(g, _):
                h1 = []
                for j in range(_H1):
                    v = h1_v[pl.ds(g * _H1 * _L + j * _L, _L)]
                    h1.append(jnp.maximum(v + b1v[j], 0.0))
                h2 = []
                for k in range(_H2):
                    wv2 = w_v[pl.ds(_OW2 + k * _H1, _L)]
                    a = zero
                    for j in range(_H1):
                        a = a + h1[j] * wv2[j]
                    h2.append(jnp.maximum(a + b2v[k], 0.0))
                o = zero
                for k in range(_H2):
                    o = o + h2[k] * w3v[k]
                bacc = h1_v[pl.ds((_G * _H1 + g) * _L, _L)]
                o = o + w3v[_OB3 - _OW3] + bacc
                out_v[pl.ds(out_off + g * _L, _L)] = o
                return 0

            lax.fori_loop(0, _G, g_body, 0, unroll=1)

        start_gather_dyn(0, 0)
        start_gather_dyn(1, 1)

        def pair_body(p, _):
            c0 = 2 * p
            wait_gather(0)
            layer1(r0_v)

            @pl.when(p < (n_chunks // 2) - 1)
            def _():
                start_gather_dyn(c0 + 2, 0)

            tail(c0 * _SPB)
            wait_gather(1)
            layer1(r1_v)

            @pl.when(p < (n_chunks // 2) - 1)
            def _():
                start_gather_dyn(c0 + 3, 1)

            tail((c0 + 1) * _SPB)
            return 0

        lax.fori_loop(0, n_chunks // 2, pair_body, 0, unroll=1)

        pltpu.sync_copy(out_v, out_hbm.at[pl.ds(sbase, spw)])

    return fused_k


def kernel(fids_batch, emb_w, emb_b, W1, b1, W2, b2, W3, b3):
    B, F = fids_batch.shape
    V, D = emb_w.shape
    N = B * F

    tab = jnp.concatenate(
        [emb_w, emb_b[:, None], jnp.zeros((V, _RW - D - 1), jnp.float32)],
        axis=1)
    fids_flat = fids_batch.reshape(N)

    wpack = jnp.concatenate([
        W1.T.reshape(F * D * _H1),
        b1,
        W2.reshape(_H2 * _H1),
        b2,
        W3.reshape(_H2),
        b3,
        jnp.zeros((_WLEN - _OB3 - 1,), jnp.float32),
    ])

    return _make_fused(B, N)(tab, fids_flat, wpack)


# speedup vs baseline: 1.3330x; 1.3330x over previous
import functools

import jax
import jax.numpy as jnp
from jax import lax
from jax.experimental import pallas as pl
from jax.experimental.pallas import tpu as pltpu
from jax.experimental.pallas import tpu_sc as plsc

_NC = 2
_NS = 16
_L = 16
_F = 66
_D = 4
_RW = 8
_H1 = 16
_H2 = 8
_SPB = 64
_G = _SPB // _L

_OW1 = 0
_OB1 = _OW1 + _F * _D * _D * _L
_OW2 = _OB1 + _H1
_OB2 = _OW2 + _H2 * _H1
_OW3 = _OB2 + _H2
_OB3 = _OW3 + _H2
_WLEN = 17064


@functools.lru_cache(maxsize=None)
def _make_fused(B, n_idx):
    nw = _NC * _NS
    spw = B // nw
    n_chunks = spw // _SPB
    ch = _SPB * _F
    assert spw % _SPB == 0 and ch % 8 == 0 and n_chunks % 2 == 0

    mesh = plsc.VectorSubcoreMesh(
        core_axis_name="c", subcore_axis_name="s",
        num_cores=_NC, num_subcores=_NS)

    @functools.partial(
        pl.kernel,
        out_type=jax.ShapeDtypeStruct((B,), jnp.float32),
        mesh=mesh,
        scratch_types=[
            pltpu.VMEM((ch,), jnp.int32),
            pltpu.VMEM((ch,), jnp.int32),
            pltpu.VMEM((ch, _RW), jnp.float32),
            pltpu.VMEM((ch, _RW), jnp.float32),
            pltpu.VMEM((_WLEN,), jnp.float32),
            pltpu.VMEM(((_G * _H1 + _G) * _L,), jnp.float32),
            pltpu.VMEM((spw,), jnp.float32),
            pltpu.SemaphoreType.DMA((2,)),
        ],
        compiler_params=pltpu.CompilerParams(
            use_tc_tiling_on_sc=False, needs_layout_passes=False),
    )
    def fused_k(tab_hbm, idx_hbm, wpack_hbm, out_hbm,
                i0_v, i1_v, r0_v, r1_v, w_v, h1_v, out_v, gsem):
        wid = lax.axis_index("s") * _NC + lax.axis_index("c")
        sbase = wid * spw
        ibase = wid * spw * _F
        idx_bufs = (i0_v, i1_v)
        row_bufs = (r0_v, r1_v)

        pltpu.sync_copy(wpack_hbm, w_v)

        iota = lax.iota(jnp.int32, _L)
        rep = [((iota // 4) + sg * 4) * _F for sg in range(_D)]
        rowb = iota * _F
        tailb = (iota // 4) * (_D * _L) + (iota % 4) * _D
        dcol = [jnp.full((_L,), d, jnp.int32) for d in range(_D + 1)]
        zero = jnp.zeros((_L,), jnp.float32)

        def start_gather_dyn(c_off, parity):
            pltpu.sync_copy(
                idx_hbm.at[pl.ds(ibase + c_off * ch, ch)], idx_bufs[parity])
            pltpu.async_copy(
                tab_hbm.at[idx_bufs[parity]], row_bufs[parity],
                gsem.at[parity])

        def wait_gather(parity):
            pltpu.make_async_copy(
                tab_hbm.at[pl.ds(0, ch)], row_bufs[parity],
                gsem.at[parity]).wait()

        def layer1(rv):
            def g_body(g, _, rv=rv):
                goff = g * _L * _F

                def f_body(f, carry):
                    accs = list(carry[:16])
                    bacc = carry[16]
                    base = goff + f
                    xs = [[plsc.load_gather(rv, [rep[sg] + base, dcol[d]])
                           for d in range(_D)] for sg in range(_D)]
                    wbase = (f * _D) * _L * _D
                    for d in range(_D):
                        for jb in range(_D):
                            wv = w_v[pl.ds(_OW1 + wbase + (d * _D + jb) * _L,
                                           _L)]
                            for sg in range(_D):
                                accs[sg * _D + jb] = (
                                    accs[sg * _D + jb] + xs[sg][d] * wv)
                    bacc = bacc + plsc.load_gather(
                        rv, [rowb + base, dcol[_D]])
                    return tuple(accs) + (bacc,)

                out = lax.fori_loop(
                    0, _F, f_body, (zero,) * 16 + (zero,), unroll=1)
                for sg in range(_D):
                    for jb in range(_D):
                        h1_v[pl.ds((g * _L + sg * _D + jb) * _L, _L)] = (
                            out[sg * _D + jb])
                h1_v[pl.ds((_G * _H1 + g) * _L, _L)] = out[16]
                return 0

            lax.fori_loop(0, _G, g_body, 0, unroll=1)

        def tail(out_off):
            b1v = w_v[pl.ds(_OB1, _L)]
            b2v = w_v[pl.ds(_OB2, _L)]
            w3v = w_v[pl.ds(_OW3, _L)]

            def g_body(g, _):
                h1 = []
                for jb in range(_D):
                    for u in range(_D):
                        idx = tailb + (g * (_H1 * _L) + jb * _L + u)
                        v = plsc.load_gather(h1_v, [idx])
                        h1.append(jnp.maximum(v + b1v[jb * _D + u], 0.0))
                h2 = []
                for k in range(_H2):
                    wv2 = w_v[pl.ds(_OW2 + k * _H1, _L)]
                    a = zero
                    for j in range(_H1):
                        a = a + h1[j] * wv2[j]
                    h2.append(jnp.maximum(a + b2v[k], 0.0))
                o = zero
                for k in range(_H2):
                    o = o + h2[k] * w3v[k]
                bacc = h1_v[pl.ds((_G * _H1 + g) * _L, _L)]
                o = o + w3v[_OB3 - _OW3] + bacc
                out_v[pl.ds(out_off + g * _L, _L)] = o
                return 0

            lax.fori_loop(0, _G, g_body, 0, unroll=1)

        start_gather_dyn(0, 0)
        start_gather_dyn(1, 1)

        def pair_body(p, _):
            c0 = 2 * p
            wait_gather(0)
            layer1(r0_v)

            @pl.when(p < (n_chunks // 2) - 1)
            def _():
                start_gather_dyn(c0 + 2, 0)

            tail(c0 * _SPB)
            wait_gather(1)
            layer1(r1_v)

            @pl.when(p < (n_chunks // 2) - 1)
            def _():
                start_gather_dyn(c0 + 3, 1)

            tail((c0 + 1) * _SPB)
            return 0

        lax.fori_loop(0, n_chunks // 2, pair_body, 0, unroll=1)

        pltpu.sync_copy(out_v, out_hbm.at[pl.ds(sbase, spw)])

    return fused_k


def kernel(fids_batch, emb_w, emb_b, W1, b1, W2, b2, W3, b3):
    B, F = fids_batch.shape
    V, D = emb_w.shape
    N = B * F

    tab = jnp.concatenate(
        [emb_w, emb_b[:, None], jnp.zeros((V, _RW - D - 1), jnp.float32)],
        axis=1)
    fids_flat = fids_batch.reshape(N)

    W1T = W1.T
    wrep = jnp.broadcast_to(
        W1T.reshape(F * D, _D, 1, _D), (F * D, _D, _D, _D)).reshape(-1)

    wpack = jnp.concatenate([
        wrep,
        b1,
        W2.reshape(_H2 * _H1),
        b2,
        W3.reshape(_H2),
        b3,
        jnp.zeros((_WLEN - _OB3 - 1,), jnp.float32),
    ])

    return _make_fused(B, N)(tab, fids_flat, wpack)
